# baseline (speedup 1.0000x reference)
import functools

import jax
import jax.numpy as jnp
from jax import lax
from jax.experimental import pallas as pl
from jax.experimental.pallas import tpu as pltpu
from jax.experimental.pallas import tpu_sc as plsc

B, N, DIM = 8, 4096, 3
D = 256
SUB, LANE = 32, 128
NW = 32
ROWS_W = (B * 2 * N) // NW
PADW = 16


def _hilbert_d(g):
    n = 3
    X = list(g)
    Q = 128
    while Q > 1:
        P = Q - 1
        for i in range(n):
            cond = (X[i] & Q) > 0
            t = (X[0] ^ X[i]) & P
            new_x0 = jnp.where(cond, X[0] ^ P, X[0] ^ t)
            new_xi = jnp.where(cond, X[i], X[i] ^ t)
            X[0] = new_x0
            if i != 0:
                X[i] = new_xi
        Q >>= 1
    for i in range(1, n):
        X[i] = X[i] ^ X[i - 1]
    t = jnp.zeros_like(X[0])
    Q = 128
    while Q > 1:
        cond = (X[n - 1] & Q) > 0
        t = jnp.where(cond, t ^ (Q - 1), t)
        Q >>= 1
    X = [x ^ t for x in X]
    h = jnp.zeros_like(X[0])
    for bit in range(7, -1, -1):
        for i in range(n):
            h = (h << 1) | ((X[i] >> bit) & 1)
    return h


def _keys_kernel(xT_ref, d_ref):
    gs = []
    for i in range(3):
        ci = xT_ref[0, i]
        mn = jnp.min(ci)
        mx = jnp.max(ci)
        span = jnp.clip(mx - mn, 1e-6, None)
        nrm = (ci - mn) / span
        gs.append(jnp.clip(nrm * 255.0, 0.0, 255.0).astype(jnp.int32))
    blk = lax.broadcasted_iota(jnp.int32, (SUB, LANE), 0)
    d_ref[0, 0] = _hilbert_d([gs[0], gs[1], gs[2]]) * SUB + blk
    d_ref[0, 1] = _hilbert_d([gs[2], gs[1], gs[0]]) * SUB + blk


def _keys_call(xT):
    return pl.pallas_call(
        _keys_kernel,
        grid=(B,),
        in_specs=[pl.BlockSpec((1, 3, SUB, LANE), lambda c: (c, 0, 0, 0))],
        out_specs=pl.BlockSpec((1, 2, SUB, LANE), lambda c: (c, 0, 0, 0)),
        out_shape=jax.ShapeDtypeStruct((B, 2, SUB, LANE), jnp.int32),
    )(xT)


def _tcol(eye, row_f):
    return jax.lax.dot_general(
        eye, row_f, (((1,), (1,)), ((), ())),
        precision=jax.lax.Precision.HIGHEST,
        preferred_element_type=jnp.float32,
    )


def _rank_kernel(krow_ref, kdiag_ref, eye_ref, dest_ref):
    c = pl.program_id(0)
    o = pl.program_id(1)
    kr = krow_ref[0, 0]
    kd = kdiag_ref[0, 0, 0]
    eye = eye_ref[...]
    hi = _tcol(eye, (kd >> 15).astype(jnp.float32))
    lo = _tcol(eye, (kd & 0x7FFF).astype(jnp.float32))
    kc = hi.astype(jnp.int32) * 32768 + lo.astype(jnp.int32)
    acc = jnp.zeros((128, 128), jnp.int32)
    for jc in range(N // 128):
        krc = kr[:, jc * 128 : (jc + 1) * 128]
        acc = acc + jnp.where(krc < kc, 1, 0)
    jlt = lax.broadcasted_iota(jnp.int32, (128, 128), 1) < lax.broadcasted_iota(
        jnp.int32, (128, 128), 0
    )
    acc = acc + jnp.where((kd == kc) & jlt, 1, 0)
    cnt = jnp.sum(acc, axis=1, keepdims=True) + (c * 2 + o) * N
    cnt_row = jax.lax.dot_general(
        cnt.astype(jnp.float32), eye, (((0,), (0,)), ((), ())),
        precision=jax.lax.Precision.HIGHEST,
        preferred_element_type=jnp.float32,
    )
    dest_ref[0, 0, 0] = cnt_row.astype(jnp.int32)


def _rank_call(krow, kdiag, eye):
    return pl.pallas_call(
        _rank_kernel,
        grid=(B, 2, N // 128),
        in_specs=[
            pl.BlockSpec((1, 1, 1, N), lambda c, o, i: (c, o, 0, 0)),
            pl.BlockSpec((1, 1, 1, 1, 128), lambda c, o, i: (c, o, i, 0, 0)),
            pl.BlockSpec((128, 128), lambda c, o, i: (0, 0)),
        ],
        out_specs=pl.BlockSpec((1, 1, 1, 1, 128), lambda c, o, i: (c, o, i, 0, 0)),
        out_shape=jax.ShapeDtypeStruct((B, 2, SUB, 1, LANE), jnp.int32),
    )(krow, kdiag, eye)


def _scatter_call(xpad, dest):
    mesh = plsc.VectorSubcoreMesh(core_axis_name="c", subcore_axis_name="s")

    @functools.partial(
        pl.kernel,
        mesh=mesh,
        out_type=jax.ShapeDtypeStruct((B * 2 * N, PADW), jnp.float32),
        scratch_types=[
            pltpu.VMEM((ROWS_W // 128, 128), jnp.int32),
            pltpu.VMEM((ROWS_W, PADW), jnp.float32),
            pltpu.SemaphoreType.DMA,
        ],
        compiler_params=pltpu.CompilerParams(use_tc_tiling_on_sc=False),
    )
    def k(xpad_hbm, dest_hbm, out_hbm, idx_v, rows_v, sem):
        wid = lax.axis_index("s") * 2 + lax.axis_index("c")
        c = wid // 4
        o = (wid % 4) // 2
        half = wid % 2
        pltpu.sync_copy(
            dest_hbm.at[c, o, pl.ds(half * (ROWS_W // 128), ROWS_W // 128)],
            idx_v,
        )
        pltpu.sync_copy(xpad_hbm.at[c, pl.ds(half * ROWS_W, ROWS_W)], rows_v)
        cps = []
        for j in range(ROWS_W // 128):
            cps.append(
                pltpu.async_copy(
                    rows_v.at[pl.ds(j * 128, 128)],
                    out_hbm.at[idx_v.at[j]],
                    sem,
                )
            )
        for cp in cps:
            cp.wait()

    return k(xpad, dest)


_KCH = 1024


def _embed_kernel(sx_ref, wt_ref, b_ref, gam_ref, bet_ref, out_ref):
    xb = sx_ref[0, 0]
    t = (
        xb[:, 0:1] * wt_ref[0:1, :]
        + xb[:, 1:2] * wt_ref[1:2, :]
        + xb[:, 2:3] * wt_ref[2:3, :]
        + b_ref[...]
    )
    out_ref[0, 0] = t * gam_ref[0] + bet_ref[0]


def _embed_call(sx, wt, b2, gamma, beta):
    return pl.pallas_call(
        _embed_kernel,
        grid=(B, 2, N // _KCH),
        in_specs=[
            pl.BlockSpec((1, 1, _KCH, PADW), lambda c, o, k: (c, o, k, 0)),
            pl.BlockSpec((8, D), lambda c, o, k: (0, 0)),
            pl.BlockSpec((1, D), lambda c, o, k: (0, 0)),
            pl.BlockSpec((1, 1, D), lambda c, o, k: (o, 0, 0)),
            pl.BlockSpec((1, 1, D), lambda c, o, k: (o, 0, 0)),
        ],
        out_specs=pl.BlockSpec((1, 1, _KCH, D), lambda c, o, k: (c, o, k, 0)),
        out_shape=jax.ShapeDtypeStruct((B, 2, N, D), jnp.float32),
    )(sx, wt, b2, gamma.reshape(2, 1, D), beta.reshape(2, 1, D))


def kernel(x, W, b, gamma, beta):
    xT = x.transpose(0, 2, 1).reshape(B, 3, SUB, LANE)
    d = _keys_call(xT)
    dest = _rank_call(
        d.reshape(B, 2, 1, N),
        d.reshape(B, 2, SUB, 1, LANE),
        jnp.eye(128, dtype=jnp.float32),
    )
    destsc = dest.reshape(B, 2, SUB, LANE)
    xpad = jnp.pad(x, ((0, 0), (0, 0), (0, PADW - 3)))
    sx = _scatter_call(xpad, destsc)
    wt = jnp.zeros((8, D), jnp.float32).at[:3].set(W.T)
    out = _embed_call(
        sx.reshape(B, 2, N, PADW),
        wt,
        b.reshape(1, D),
        gamma,
        beta,
    )
    return out.reshape(B, 2 * N, D)


# speedup vs baseline: 1.3604x; 1.3604x over previous
import functools

import jax
import jax.numpy as jnp
from jax import lax
from jax.experimental import pallas as pl
from jax.experimental.pallas import tpu as pltpu
from jax.experimental.pallas import tpu_sc as plsc

B, N, DIM = 8, 4096, 3
D = 256
SUB, LANE = 32, 128
NW = 32
ROWS_W = (B * 2 * N) // NW
PADW = 16


def _hilbert_d(g):
    n = 3
    X = list(g)
    Q = 128
    while Q > 1:
        P = Q - 1
        for i in range(n):
            cond = (X[i] & Q) > 0
            t = (X[0] ^ X[i]) & P
            new_x0 = jnp.where(cond, X[0] ^ P, X[0] ^ t)
            new_xi = jnp.where(cond, X[i], X[i] ^ t)
            X[0] = new_x0
            if i != 0:
                X[i] = new_xi
        Q >>= 1
    for i in range(1, n):
        X[i] = X[i] ^ X[i - 1]
    t = jnp.zeros_like(X[0])
    Q = 128
    while Q > 1:
        cond = (X[n - 1] & Q) > 0
        t = jnp.where(cond, t ^ (Q - 1), t)
        Q >>= 1
    X = [x ^ t for x in X]
    h = jnp.zeros_like(X[0])
    for bit in range(7, -1, -1):
        for i in range(n):
            h = (h << 1) | ((X[i] >> bit) & 1)
    return h


def _keys_kernel(xT_ref, d_ref):
    gs = []
    for i in range(3):
        ci = xT_ref[0, i]
        mn = jnp.min(ci)
        mx = jnp.max(ci)
        span = jnp.clip(mx - mn, 1e-6, None)
        nrm = (ci - mn) / span
        gs.append(jnp.clip(nrm * 255.0, 0.0, 255.0).astype(jnp.int32))
    blk = lax.broadcasted_iota(jnp.int32, (SUB, LANE), 0)
    d_ref[0, 0] = _hilbert_d([gs[0], gs[1], gs[2]]) * SUB + blk
    d_ref[0, 1] = _hilbert_d([gs[2], gs[1], gs[0]]) * SUB + blk


def _keys_call(xT):
    return pl.pallas_call(
        _keys_kernel,
        grid=(B,),
        in_specs=[pl.BlockSpec((1, 3, SUB, LANE), lambda c: (c, 0, 0, 0))],
        out_specs=pl.BlockSpec((1, 2, SUB, LANE), lambda c: (c, 0, 0, 0)),
        out_shape=jax.ShapeDtypeStruct((B, 2, SUB, LANE), jnp.int32),
    )(xT)


def _tcol(eye, row_f):
    return jax.lax.dot_general(
        eye, row_f, (((1,), (1,)), ((), ())),
        precision=jax.lax.Precision.HIGHEST,
        preferred_element_type=jnp.float32,
    )


_IBLK = 4


def _rank_kernel(krow_ref, kdiag_ref, eye_ref, dest_ref):
    c = pl.program_id(0)
    o = pl.program_id(1)
    kr = krow_ref[0, 0]
    kd4 = kdiag_ref[0, 0, :, 0, :]
    eye = eye_ref[...]
    hi = _tcol(eye, (kd4 >> 15).astype(jnp.float32))
    lo = _tcol(eye, (kd4 & 0x7FFF).astype(jnp.float32))
    kct = hi.astype(jnp.int32) * 32768 + lo.astype(jnp.int32)
    jlt = lax.broadcasted_iota(jnp.int32, (128, 128), 1) < lax.broadcasted_iota(
        jnp.int32, (128, 128), 0
    )
    cnt_cols = []
    for s in range(_IBLK):
        kc = kct[:, s : s + 1]
        acc = jnp.where((kd4[s : s + 1, :] == kc) & jlt, 1, 0)
        for jc in range(N // 128):
            krc = kr[:, jc * 128 : (jc + 1) * 128]
            acc = acc + jnp.where(krc < kc, 1, 0)
        cnt_cols.append(jnp.sum(acc, axis=1, keepdims=True))
    cnt = jnp.concatenate(cnt_cols, axis=1) + (c * 2 + o) * N
    cnt_rows = jax.lax.dot_general(
        cnt.astype(jnp.float32), eye, (((0,), (0,)), ((), ())),
        precision=jax.lax.Precision.HIGHEST,
        preferred_element_type=jnp.float32,
    )
    dest_ref[0, 0, :, 0, :] = cnt_rows.astype(jnp.int32)


def _rank_call(krow, kdiag, eye):
    return pl.pallas_call(
        _rank_kernel,
        grid=(B, 2, N // (128 * _IBLK)),
        in_specs=[
            pl.BlockSpec((1, 1, 1, N), lambda c, o, i: (c, o, 0, 0)),
            pl.BlockSpec((1, 1, _IBLK, 1, 128), lambda c, o, i: (c, o, i, 0, 0)),
            pl.BlockSpec((128, 128), lambda c, o, i: (0, 0)),
        ],
        out_specs=pl.BlockSpec(
            (1, 1, _IBLK, 1, 128), lambda c, o, i: (c, o, i, 0, 0)
        ),
        out_shape=jax.ShapeDtypeStruct((B, 2, SUB, 1, LANE), jnp.int32),
    )(krow, kdiag, eye)


def _scatter_call(xpad, dest):
    mesh = plsc.VectorSubcoreMesh(core_axis_name="c", subcore_axis_name="s")

    @functools.partial(
        pl.kernel,
        mesh=mesh,
        out_type=jax.ShapeDtypeStruct((B * 2 * N, PADW), jnp.float32),
        scratch_types=[
            pltpu.VMEM((ROWS_W // 128, 128), jnp.int32),
            pltpu.VMEM((ROWS_W, PADW), jnp.float32),
            pltpu.SemaphoreType.DMA,
        ],
        compiler_params=pltpu.CompilerParams(use_tc_tiling_on_sc=False),
    )
    def k(xpad_hbm, dest_hbm, out_hbm, idx_v, rows_v, sem):
        wid = lax.axis_index("s") * 2 + lax.axis_index("c")
        c = wid // 4
        o = (wid % 4) // 2
        half = wid % 2
        pltpu.sync_copy(
            dest_hbm.at[c, o, pl.ds(half * (ROWS_W // 128), ROWS_W // 128)],
            idx_v,
        )
        pltpu.sync_copy(xpad_hbm.at[c, pl.ds(half * ROWS_W, ROWS_W)], rows_v)
        cps = []
        for j in range(ROWS_W // 128):
            cps.append(
                pltpu.async_copy(
                    rows_v.at[pl.ds(j * 128, 128)],
                    out_hbm.at[idx_v.at[j]],
                    sem,
                )
            )
        for cp in cps:
            cp.wait()

    return k(xpad, dest)


_KCH = 1024


def _embed_kernel(sx_ref, wt_ref, b_ref, out_ref):
    xb = sx_ref[0, 0]
    out_ref[0, 0] = (
        xb[:, 0:1] * wt_ref[0, 0:1, :]
        + xb[:, 1:2] * wt_ref[0, 1:2, :]
        + xb[:, 2:3] * wt_ref[0, 2:3, :]
        + b_ref[0]
    )


def _embed_call(sx, wt2, b2):
    return pl.pallas_call(
        _embed_kernel,
        grid=(B, 2, N // _KCH),
        in_specs=[
            pl.BlockSpec((1, 1, _KCH, PADW), lambda c, o, k: (c, o, k, 0)),
            pl.BlockSpec((1, 8, D), lambda c, o, k: (o, 0, 0)),
            pl.BlockSpec((1, 1, D), lambda c, o, k: (o, 0, 0)),
        ],
        out_specs=pl.BlockSpec((1, 1, _KCH, D), lambda c, o, k: (c, o, k, 0)),
        out_shape=jax.ShapeDtypeStruct((B, 2, N, D), jnp.float32),
    )(sx, wt2, b2)


def kernel(x, W, b, gamma, beta):
    xT = x.transpose(0, 2, 1).reshape(B, 3, SUB, LANE)
    d = _keys_call(xT)
    dest = _rank_call(
        d.reshape(B, 2, 1, N),
        d.reshape(B, 2, SUB, 1, LANE),
        jnp.eye(128, dtype=jnp.float32),
    )
    destsc = dest.reshape(B, 2, SUB, LANE)
    xpad = jnp.pad(x, ((0, 0), (0, 0), (0, PADW - 3)))
    sx = _scatter_call(xpad, destsc)
    wt2 = jnp.zeros((2, 8, D), jnp.float32).at[:, :3].set(
        W.T[None] * gamma[:, None, :]
    )
    b2 = (b[None] * gamma + beta).reshape(2, 1, D)
    out = _embed_call(sx.reshape(B, 2, N, PADW), wt2, b2)
    return out.reshape(B, 2 * N, D)


# speedup vs baseline: 1.5410x; 1.1327x over previous
import functools

import jax
import jax.numpy as jnp
from jax import lax
from jax.experimental import pallas as pl
from jax.experimental.pallas import tpu as pltpu
from jax.experimental.pallas import tpu_sc as plsc

B, N, DIM = 8, 4096, 3
D = 256
SUB, LANE = 32, 128
NW = 32
ROWS_W = (B * 2 * N) // NW
PADW = 16


def _hilbert_d(g):
    n = 3
    X = list(g)
    Q = 128
    while Q > 1:
        P = Q - 1
        for i in range(n):
            cond = (X[i] & Q) > 0
            t = (X[0] ^ X[i]) & P
            new_x0 = jnp.where(cond, X[0] ^ P, X[0] ^ t)
            new_xi = jnp.where(cond, X[i], X[i] ^ t)
            X[0] = new_x0
            if i != 0:
                X[i] = new_xi
        Q >>= 1
    for i in range(1, n):
        X[i] = X[i] ^ X[i - 1]
    t = jnp.zeros_like(X[0])
    Q = 128
    while Q > 1:
        cond = (X[n - 1] & Q) > 0
        t = jnp.where(cond, t ^ (Q - 1), t)
        Q >>= 1
    X = [x ^ t for x in X]
    h = jnp.zeros_like(X[0])
    for bit in range(7, -1, -1):
        for i in range(n):
            h = (h << 1) | ((X[i] >> bit) & 1)
    return h


def _keys_kernel(xT_ref, d_ref):
    gs = []
    for i in range(3):
        ci = xT_ref[0, i]
        mn = jnp.min(ci)
        mx = jnp.max(ci)
        span = jnp.clip(mx - mn, 1e-6, None)
        nrm = (ci - mn) / span
        gs.append(jnp.clip(nrm * 255.0, 0.0, 255.0).astype(jnp.int32))
    blk = lax.broadcasted_iota(jnp.int32, (SUB, LANE), 0)
    d_ref[0, 0] = _hilbert_d([gs[0], gs[1], gs[2]]) * SUB + blk
    d_ref[0, 1] = _hilbert_d([gs[2], gs[1], gs[0]]) * SUB + blk


def _keys_call(xT):
    return pl.pallas_call(
        _keys_kernel,
        grid=(B,),
        in_specs=[pl.BlockSpec((1, 3, SUB, LANE), lambda c: (c, 0, 0, 0))],
        out_specs=pl.BlockSpec((1, 2, SUB, LANE), lambda c: (c, 0, 0, 0)),
        out_shape=jax.ShapeDtypeStruct((B, 2, SUB, LANE), jnp.int32),
    )(xT)


def _tcol(eye, row_f):
    return jax.lax.dot_general(
        eye, row_f, (((1,), (1,)), ((), ())),
        precision=jax.lax.Precision.HIGHEST,
        preferred_element_type=jnp.float32,
    )


_IBLK = 8


def _rank_kernel(krow_ref, kdiag_ref, eye_ref, dest_ref):
    c = pl.program_id(0)
    o = pl.program_id(1)
    kr = krow_ref[0, 0]
    kd4 = kdiag_ref[0, 0, :, 0, :]
    eye = eye_ref[...]
    hi = _tcol(eye, (kd4 >> 15).astype(jnp.float32))
    lo = _tcol(eye, (kd4 & 0x7FFF).astype(jnp.float32))
    kct = hi.astype(jnp.int32) * 32768 + lo.astype(jnp.int32)
    jlt = lax.broadcasted_iota(jnp.int32, (128, 128), 1) < lax.broadcasted_iota(
        jnp.int32, (128, 128), 0
    )
    cnt_cols = []
    for s in range(_IBLK):
        kc = kct[:, s : s + 1]
        acc = jnp.where((kd4[s : s + 1, :] == kc) & jlt, 1, 0)
        for jc in range(N // 128):
            krc = kr[:, jc * 128 : (jc + 1) * 128]
            acc = acc + jnp.where(krc < kc, 1, 0)
        cnt_cols.append(jnp.sum(acc, axis=1, keepdims=True))
    cnt = jnp.concatenate(cnt_cols, axis=1) + (c * 2 + o) * N
    cnt_rows = jax.lax.dot_general(
        cnt.astype(jnp.float32), eye, (((0,), (0,)), ((), ())),
        precision=jax.lax.Precision.HIGHEST,
        preferred_element_type=jnp.float32,
    )
    dest_ref[0, 0, :, 0, :] = cnt_rows.astype(jnp.int32)


def _rank_call(krow, kdiag, eye):
    return pl.pallas_call(
        _rank_kernel,
        grid=(B, 2, N // (128 * _IBLK)),
        in_specs=[
            pl.BlockSpec((1, 1, 1, N), lambda c, o, i: (c, o, 0, 0)),
            pl.BlockSpec((1, 1, _IBLK, 1, 128), lambda c, o, i: (c, o, i, 0, 0)),
            pl.BlockSpec((128, 128), lambda c, o, i: (0, 0)),
        ],
        out_specs=pl.BlockSpec(
            (1, 1, _IBLK, 1, 128), lambda c, o, i: (c, o, i, 0, 0)
        ),
        out_shape=jax.ShapeDtypeStruct((B, 2, SUB, 1, LANE), jnp.int32),
    )(krow, kdiag, eye)


def _scatter_call(xpad, dest):
    mesh = plsc.VectorSubcoreMesh(core_axis_name="c", subcore_axis_name="s")

    @functools.partial(
        pl.kernel,
        mesh=mesh,
        out_type=jax.ShapeDtypeStruct((B * 2 * N, PADW), jnp.float32),
        scratch_types=[
            pltpu.VMEM((ROWS_W // 128, 128), jnp.int32),
            pltpu.VMEM((ROWS_W, PADW), jnp.float32),
            pltpu.SemaphoreType.DMA,
        ],
        compiler_params=pltpu.CompilerParams(use_tc_tiling_on_sc=False),
    )
    def k(xpad_hbm, dest_hbm, out_hbm, idx_v, rows_v, sem):
        wid = lax.axis_index("s") * 2 + lax.axis_index("c")
        c = wid // 4
        o = (wid % 4) // 2
        half = wid % 2
        pltpu.sync_copy(
            dest_hbm.at[c, o, pl.ds(half * (ROWS_W // 128), ROWS_W // 128)],
            idx_v,
        )
        pltpu.sync_copy(xpad_hbm.at[c, pl.ds(half * ROWS_W, ROWS_W)], rows_v)
        cps = []
        for j in range(ROWS_W // 128):
            cps.append(
                pltpu.async_copy(
                    rows_v.at[pl.ds(j * 128, 128)],
                    out_hbm.at[idx_v.at[j]],
                    sem,
                )
            )
        for cp in cps:
            cp.wait()

    return k(xpad, dest)


_KCH = 1024


def _embed_kernel(sx_ref, wt_ref, b_ref, out_ref):
    xb = sx_ref[0, 0]
    out_ref[0, 0] = (
        xb[:, 0:1] * wt_ref[0, 0:1, :]
        + xb[:, 1:2] * wt_ref[0, 1:2, :]
        + xb[:, 2:3] * wt_ref[0, 2:3, :]
        + b_ref[0]
    )


def _embed_call(sx, wt2, b2):
    return pl.pallas_call(
        _embed_kernel,
        grid=(B, 2, N // _KCH),
        in_specs=[
            pl.BlockSpec((1, 1, _KCH, PADW), lambda c, o, k: (c, o, k, 0)),
            pl.BlockSpec((1, 8, D), lambda c, o, k: (o, 0, 0)),
            pl.BlockSpec((1, 1, D), lambda c, o, k: (o, 0, 0)),
        ],
        out_specs=pl.BlockSpec((1, 1, _KCH, D), lambda c, o, k: (c, o, k, 0)),
        out_shape=jax.ShapeDtypeStruct((B, 2, N, D), jnp.float32),
    )(sx, wt2, b2)


def kernel(x, W, b, gamma, beta):
    xT = x.transpose(0, 2, 1).reshape(B, 3, SUB, LANE)
    d = _keys_call(xT)
    dest = _rank_call(
        d.reshape(B, 2, 1, N),
        d.reshape(B, 2, SUB, 1, LANE),
        jnp.eye(128, dtype=jnp.float32),
    )
    destsc = dest.reshape(B, 2, SUB, LANE)
    xpad = jnp.pad(x, ((0, 0), (0, 0), (0, PADW - 3)))
    sx = _scatter_call(xpad, destsc)
    wt2 = jnp.zeros((2, 8, D), jnp.float32).at[:, :3].set(
        W.T[None] * gamma[:, None, :]
    )
    b2 = (b[None] * gamma + beta).reshape(2, 1, D)
    out = _embed_call(sx.reshape(B, 2, N, PADW), wt2, b2)
    return out.reshape(B, 2 * N, D)
